# Initial kernel scaffold; baseline (speedup 1.0000x reference)
import jax
import jax.numpy as jnp
from jax.experimental import pallas as pl


def kernel(router_logits, correction_bias):
    raise NotImplementedError("write your pallas kernel here")


# speedup vs baseline: 29.2785x; 29.2785x over previous
import functools

import jax
import jax.numpy as jnp
from jax import lax
from jax.experimental import pallas as pl
from jax.experimental.pallas import tpu as pltpu
from jax.experimental.pallas import tpu_sc as plsc

TOP_K = 8
N_EXPERTS = 256
SCALING = 2.5
N_GROUP = 8
TOPK_GROUP = 4
GROUP_SIZE = N_EXPERTS // N_GROUP

NC, NS, L = 2, 16, 16
NW = NC * NS
N_SUPER = N_EXPERTS // L
NEG = -1e9


def _router_body(logits_hbm, bias_hbm, idx_hbm, w_hbm, xbuf, st, sup, bias_v,
                 oi, ow):
    n_tokens = logits_hbm.shape[0]
    tok_per_w = n_tokens // NW
    n_tiles = tok_per_w // L

    wid = lax.axis_index("s") * NC + lax.axis_index("c")
    lane = lax.iota(jnp.int32, L)
    negv = jnp.full((L,), NEG, jnp.float32)
    zero_i = jnp.zeros((L,), jnp.int32)

    pltpu.sync_copy(bias_hbm, bias_v)

    def tile_body(t, _):
        base = wid * tok_per_w + t * L
        pltpu.sync_copy(logits_hbm.at[pl.ds(base, L), :], xbuf)

        g_sc = []
        for g in range(N_GROUP):
            def e_body(i, c, g=g):
                m1, m2 = c
                e = g * GROUP_SIZE + i
                ev = jnp.full((L,), 1, jnp.int32) * e
                x = plsc.load_gather(xbuf, [lane, ev])
                s = 1.0 / (1.0 + jnp.exp(-x))
                b = plsc.load_gather(bias_v, [ev])
                adj = s + b
                plsc.store_scatter(st, [ev * L + lane], adj)
                gt = adj > m1
                m2 = jnp.where(gt, m1, jnp.maximum(m2, adj))
                m1 = jnp.maximum(m1, adj)
                return m1, m2

            m1, m2 = lax.fori_loop(0, GROUP_SIZE, e_body, (negv, negv))
            g_sc.append(m1 + m2)

        chosen = [None] * N_GROUP
        for _ in range(TOPK_GROUP):
            best = g_sc[0]
            for g in range(1, N_GROUP):
                best = jnp.maximum(best, g_sc[g])
            found = lane < 0
            for g in range(N_GROUP):
                hit = (g_sc[g] == best) & (~found)
                chosen[g] = hit if chosen[g] is None else (chosen[g] | hit)
                found = found | hit
                g_sc[g] = jnp.where(hit, negv, g_sc[g])
        gadd = [jnp.where(chosen[g], 0.0, negv) for g in range(N_GROUP)]

        for s_i in range(N_SUPER):
            m = negv
            for j in range(L):
                m = jnp.maximum(m, st[pl.ds((s_i * L + j) * L, L)])
            m = m + gadd[s_i // (N_SUPER // N_GROUP)]
            sup[pl.ds(s_i * L, L)] = m

        def round_body(r, wsum):
            m = negv
            mi = zero_i
            for s_i in range(N_SUPER):
                v = sup[pl.ds(s_i * L, L)]
                gt = v > m
                m = jnp.where(gt, v, m)
                mi = jnp.where(gt, jnp.full((L,), 1, jnp.int32) * s_i, mi)
            lm = negv
            lj = zero_i
            for j in range(L):
                v = plsc.load_gather(st, [(mi * L + j) * L + lane])
                gt = v > lm
                lm = jnp.where(gt, v, lm)
                lj = jnp.where(gt, jnp.full((L,), 1, jnp.int32) * j, lj)
            eidx = mi * L + lj
            plsc.store_scatter(st, [eidx * L + lane], negv)
            nm = negv
            for j in range(L):
                nm = jnp.maximum(nm, plsc.load_gather(st,
                                                      [(mi * L + j) * L + lane]))
            plsc.store_scatter(sup, [mi * L + lane], nm)
            b = plsc.load_gather(bias_v, [eidx])
            w = lm - b
            rv = jnp.full((L,), 1, jnp.int32) * r
            plsc.store_scatter(oi, [lane, rv], eidx)
            plsc.store_scatter(ow, [lane, rv], w)
            return wsum + w

        wsum = lax.fori_loop(0, TOP_K, round_body, jnp.zeros((L,), jnp.float32))
        scale = SCALING / (wsum + 1e-20)
        for r in range(TOP_K):
            rv = jnp.full((L,), 1, jnp.int32) * r
            w = plsc.load_gather(ow, [lane, rv])
            plsc.store_scatter(ow, [lane, rv], w * scale)

        pltpu.sync_copy(oi, idx_hbm.at[pl.ds(base, L), :])
        pltpu.sync_copy(ow, w_hbm.at[pl.ds(base, L), :])
        return 0

    lax.fori_loop(0, n_tiles, tile_body, 0)


def kernel(router_logits, correction_bias):
    n_tokens = router_logits.shape[0]
    mesh = plsc.VectorSubcoreMesh(core_axis_name="c", subcore_axis_name="s",
                                  num_cores=NC, num_subcores=NS)
    run = pl.kernel(
        _router_body,
        out_type=(
            jax.ShapeDtypeStruct((n_tokens, TOP_K), jnp.int32),
            jax.ShapeDtypeStruct((n_tokens, TOP_K), jnp.float32),
        ),
        mesh=mesh,
        scratch_types=[
            pltpu.VMEM((L, N_EXPERTS), jnp.float32),
            pltpu.VMEM((N_EXPERTS * L,), jnp.float32),
            pltpu.VMEM((N_SUPER * L,), jnp.float32),
            pltpu.VMEM((N_EXPERTS,), jnp.float32),
            pltpu.VMEM((L, TOP_K), jnp.int32),
            pltpu.VMEM((L, TOP_K), jnp.float32),
        ],
        compiler_params=pltpu.CompilerParams(use_tc_tiling_on_sc=False,
                                             needs_layout_passes=False),
    )
    return run(router_logits, correction_bias)


# speedup vs baseline: 43.0744x; 1.4712x over previous
import functools

import jax
import jax.numpy as jnp
from jax import lax
from jax.experimental import pallas as pl
from jax.experimental.pallas import tpu as pltpu
from jax.experimental.pallas import tpu_sc as plsc

TOP_K = 8
N_EXPERTS = 256
SCALING = 2.5
N_GROUP = 8
TOPK_GROUP = 4
GROUP_SIZE = N_EXPERTS // N_GROUP

NC, NS, L = 2, 16, 16
NW = NC * NS
N_SUPER = N_EXPERTS // L
NEG = -1e9


def _router_body(logits_hbm, bias_hbm, idx_hbm, w_hbm, xbuf, st, sup, bias_v,
                 oi, ow):
    n_tokens = logits_hbm.shape[0]
    tok_per_w = n_tokens // NW
    n_tiles = tok_per_w // L

    wid = lax.axis_index("s") * NC + lax.axis_index("c")
    lane = lax.iota(jnp.int32, L)
    negv = jnp.full((L,), NEG, jnp.float32)
    zero_i = jnp.zeros((L,), jnp.int32)

    pltpu.sync_copy(bias_hbm, bias_v)

    def tile_body(t, _):
        base = wid * tok_per_w + t * L
        pltpu.sync_copy(logits_hbm.at[pl.ds(base, L), :], xbuf)

        g_sc = []
        for g in range(N_GROUP):
            def e_body(i, c, g=g):
                m1, m2 = c
                e = g * GROUP_SIZE + i
                ev = jnp.full((L,), 1, jnp.int32) * e
                s = plsc.load_gather(xbuf, [lane, ev])
                b = plsc.load_gather(bias_v, [ev])
                adj = s + b
                plsc.store_scatter(st, [ev * L + lane], adj)
                gt = adj > m1
                m2 = jnp.where(gt, m1, jnp.maximum(m2, adj))
                m1 = jnp.maximum(m1, adj)
                return m1, m2

            m1, m2 = lax.fori_loop(0, GROUP_SIZE, e_body, (negv, negv))
            g_sc.append(m1 + m2)

        chosen = [None] * N_GROUP
        for _ in range(TOPK_GROUP):
            best = g_sc[0]
            for g in range(1, N_GROUP):
                best = jnp.maximum(best, g_sc[g])
            found = lane < 0
            for g in range(N_GROUP):
                hit = (g_sc[g] == best) & (~found)
                chosen[g] = hit if chosen[g] is None else (chosen[g] | hit)
                found = found | hit
                g_sc[g] = jnp.where(hit, negv, g_sc[g])
        gadd = [jnp.where(chosen[g], 0.0, negv) for g in range(N_GROUP)]

        for s_i in range(N_SUPER):
            m = negv
            for j in range(L):
                m = jnp.maximum(m, st[pl.ds((s_i * L + j) * L, L)])
            m = m + gadd[s_i // (N_SUPER // N_GROUP)]
            sup[pl.ds(s_i * L, L)] = m

        def round_body(r, wsum):
            m = negv
            mi = zero_i
            for s_i in range(N_SUPER):
                v = sup[pl.ds(s_i * L, L)]
                gt = v > m
                m = jnp.where(gt, v, m)
                mi = jnp.where(gt, jnp.full((L,), 1, jnp.int32) * s_i, mi)
            lm = negv
            lj = zero_i
            for j in range(L):
                v = plsc.load_gather(st, [(mi * L + j) * L + lane])
                gt = v > lm
                lm = jnp.where(gt, v, lm)
                lj = jnp.where(gt, jnp.full((L,), 1, jnp.int32) * j, lj)
            eidx = mi * L + lj
            plsc.store_scatter(st, [eidx * L + lane], negv)
            nm = negv
            for j in range(L):
                nm = jnp.maximum(nm, plsc.load_gather(st,
                                                      [(mi * L + j) * L + lane]))
            plsc.store_scatter(sup, [mi * L + lane], nm)
            b = plsc.load_gather(bias_v, [eidx])
            w = lm - b
            rv = jnp.full((L,), 1, jnp.int32) * r
            plsc.store_scatter(oi, [lane, rv], eidx)
            plsc.store_scatter(ow, [lane, rv], w)
            return wsum + w

        wsum = lax.fori_loop(0, TOP_K, round_body, jnp.zeros((L,), jnp.float32))
        scale = SCALING / (wsum + 1e-20)
        for r in range(TOP_K):
            rv = jnp.full((L,), 1, jnp.int32) * r
            w = plsc.load_gather(ow, [lane, rv])
            plsc.store_scatter(ow, [lane, rv], w * scale)

        pltpu.sync_copy(oi, idx_hbm.at[pl.ds(base, L), :])
        pltpu.sync_copy(ow, w_hbm.at[pl.ds(base, L), :])
        return 0

    lax.fori_loop(0, n_tiles, tile_body, 0)


def kernel(router_logits, correction_bias):
    scores = jax.nn.sigmoid(router_logits).astype(jnp.float32)
    n_tokens = router_logits.shape[0]
    mesh = plsc.VectorSubcoreMesh(core_axis_name="c", subcore_axis_name="s",
                                  num_cores=NC, num_subcores=NS)
    run = pl.kernel(
        _router_body,
        out_type=(
            jax.ShapeDtypeStruct((n_tokens, TOP_K), jnp.int32),
            jax.ShapeDtypeStruct((n_tokens, TOP_K), jnp.float32),
        ),
        mesh=mesh,
        scratch_types=[
            pltpu.VMEM((L, N_EXPERTS), jnp.float32),
            pltpu.VMEM((N_EXPERTS * L,), jnp.float32),
            pltpu.VMEM((N_SUPER * L,), jnp.float32),
            pltpu.VMEM((N_EXPERTS,), jnp.float32),
            pltpu.VMEM((L, TOP_K), jnp.int32),
            pltpu.VMEM((L, TOP_K), jnp.float32),
        ],
        compiler_params=pltpu.CompilerParams(use_tc_tiling_on_sc=False,
                                             needs_layout_passes=False),
    )
    return run(scores, correction_bias)


# speedup vs baseline: 54.3085x; 1.2608x over previous
import jax
import jax.numpy as jnp
from jax import lax
from jax.experimental import pallas as pl
from jax.experimental.pallas import tpu as pltpu
from jax.experimental.pallas import tpu_sc as plsc

TOP_K = 8
N_EXPERTS = 256
SCALING = 2.5
N_GROUP = 8
TOPK_GROUP = 4
GROUP_SIZE = N_EXPERTS // N_GROUP

NC, NS, L = 2, 16, 16
NW = NC * NS
N_SUPER = N_EXPERTS // L
SUP_PER_GROUP = GROUP_SIZE // L
NEG = -1e9


def _router_body(adj_hbm, bias_hbm, idx_hbm, w_hbm, xbuf, sup, bias_v, oi, ow):
    n_tokens = adj_hbm.shape[0]
    tok_per_w = n_tokens // NW
    n_tiles = tok_per_w // L

    wid = lax.axis_index("s") * NC + lax.axis_index("c")
    lane = lax.iota(jnp.int32, L)
    negv = jnp.full((L,), NEG, jnp.float32)
    zero_i = jnp.zeros((L,), jnp.int32)

    pltpu.sync_copy(bias_hbm, bias_v)

    def tile_body(t, _):
        base = wid * tok_per_w + t * L
        pltpu.sync_copy(adj_hbm.at[pl.ds(base, L), :], xbuf)

        sm1 = [negv] * N_SUPER
        sm2 = [negv] * N_SUPER
        for e in range(N_EXPERTS):
            si = e // L
            ev = jnp.full((L,), e, jnp.int32)
            s = plsc.load_gather(xbuf, [lane, ev])
            gt = s > sm1[si]
            sm2[si] = jnp.where(gt, sm1[si], jnp.maximum(sm2[si], s))
            sm1[si] = jnp.maximum(sm1[si], s)

        g_sc = []
        for g in range(N_GROUP):
            a1, a2 = sm1[2 * g], sm2[2 * g]
            b1, b2 = sm1[2 * g + 1], sm2[2 * g + 1]
            hi = jnp.maximum(a1, b1)
            lo = jnp.minimum(a1, b1)
            sec = jnp.maximum(lo, jnp.where(a1 > b1, a2, b2))
            g_sc.append(hi + sec)

        chosen = [None] * N_GROUP
        for _ in range(TOPK_GROUP):
            best = g_sc[0]
            for g in range(1, N_GROUP):
                best = jnp.maximum(best, g_sc[g])
            found = lane < 0
            for g in range(N_GROUP):
                hit = (g_sc[g] == best) & (~found)
                chosen[g] = hit if chosen[g] is None else (chosen[g] | hit)
                found = found | hit
                g_sc[g] = jnp.where(hit, negv, g_sc[g])

        for s_i in range(N_SUPER):
            m = jnp.where(chosen[s_i // SUP_PER_GROUP], sm1[s_i], negv)
            sup[pl.ds(s_i * L, L)] = m

        def round_body(r, wsum):
            m = negv
            mi = zero_i
            for s_i in range(N_SUPER):
                v = sup[pl.ds(s_i * L, L)]
                gt = v > m
                m = jnp.where(gt, v, m)
                mi = jnp.where(gt, jnp.full((L,), s_i, jnp.int32), mi)
            miL = mi * L
            lm = negv
            lj = zero_i
            for j in range(L):
                v = plsc.load_gather(xbuf, [lane, miL + j])
                gt = v > lm
                lm = jnp.where(gt, v, lm)
                lj = jnp.where(gt, jnp.full((L,), j, jnp.int32), lj)
            eidx = miL + lj
            plsc.store_scatter(xbuf, [lane, eidx], negv)
            nm = negv
            for j in range(L):
                nm = jnp.maximum(nm, plsc.load_gather(xbuf, [lane, miL + j]))
            plsc.store_scatter(sup, [miL + lane], nm)
            b = plsc.load_gather(bias_v, [eidx])
            w = lm - b
            tok = t * L + lane
            rv = jnp.full((L,), 1, jnp.int32) * r
            plsc.store_scatter(oi, [tok, rv], eidx)
            plsc.store_scatter(ow, [tok, rv], w)
            return wsum + w

        wsum = lax.fori_loop(0, TOP_K, round_body,
                             jnp.zeros((L,), jnp.float32))
        scale = SCALING / (wsum + 1e-20)
        tok = t * L + lane
        for r in range(TOP_K):
            rv = jnp.full((L,), r, jnp.int32)
            w = plsc.load_gather(ow, [tok, rv])
            plsc.store_scatter(ow, [tok, rv], w * scale)
        return 0

    lax.fori_loop(0, n_tiles, tile_body, 0)
    pltpu.sync_copy(oi, idx_hbm.at[pl.ds(wid * tok_per_w, tok_per_w), :])
    pltpu.sync_copy(ow, w_hbm.at[pl.ds(wid * tok_per_w, tok_per_w), :])


def kernel(router_logits, correction_bias):
    scores = jax.nn.sigmoid(router_logits).astype(jnp.float32)
    adj = scores + correction_bias[None, :]
    n_tokens = router_logits.shape[0]
    tok_per_w = n_tokens // NW
    mesh = plsc.VectorSubcoreMesh(core_axis_name="c", subcore_axis_name="s",
                                  num_cores=NC, num_subcores=NS)
    run = pl.kernel(
        _router_body,
        out_type=(
            jax.ShapeDtypeStruct((n_tokens, TOP_K), jnp.int32),
            jax.ShapeDtypeStruct((n_tokens, TOP_K), jnp.float32),
        ),
        mesh=mesh,
        scratch_types=[
            pltpu.VMEM((L, N_EXPERTS), jnp.float32),
            pltpu.VMEM((N_SUPER * L,), jnp.float32),
            pltpu.VMEM((N_EXPERTS,), jnp.float32),
            pltpu.VMEM((tok_per_w, TOP_K), jnp.int32),
            pltpu.VMEM((tok_per_w, TOP_K), jnp.float32),
        ],
        compiler_params=pltpu.CompilerParams(use_tc_tiling_on_sc=False,
                                             needs_layout_passes=False),
    )
    return run(adj, correction_bias)


# speedup vs baseline: 60.8407x; 1.1203x over previous
import jax
import jax.numpy as jnp
from jax import lax
from jax.experimental import pallas as pl
from jax.experimental.pallas import tpu as pltpu
from jax.experimental.pallas import tpu_sc as plsc

TOP_K = 8
N_EXPERTS = 256
SCALING = 2.5
N_GROUP = 8
TOPK_GROUP = 4
GROUP_SIZE = N_EXPERTS // N_GROUP

NC, NS, L = 2, 16, 16
NW = NC * NS
N_SUPER = N_EXPERTS // L
SUP_PER_GROUP = GROUP_SIZE // L
NEG = -1e9


def _router_body(adj_hbm, bias_hbm, idx_hbm, w_hbm, xbuf, sup, bias_v, oi, ow,
                 sem):
    n_tokens = adj_hbm.shape[0]
    tok_per_w = n_tokens // NW
    n_tiles = tok_per_w // L

    wid = lax.axis_index("s") * NC + lax.axis_index("c")
    lane = lax.iota(jnp.int32, L)
    negv = jnp.full((L,), NEG, jnp.float32)
    zero_i = jnp.zeros((L,), jnp.int32)

    pltpu.sync_copy(bias_hbm, bias_v)
    pltpu.async_copy(adj_hbm.at[pl.ds(wid * tok_per_w, L), :], xbuf.at[0], sem)

    def tile_body(t, _):
        base = wid * tok_per_w + t * L
        p = lax.rem(t, 2)
        pltpu.make_async_copy(adj_hbm.at[pl.ds(base, L), :], xbuf.at[p],
                              sem).wait()

        @pl.when(t < n_tiles - 1)
        def _prefetch():
            pltpu.async_copy(adj_hbm.at[pl.ds(base + L, L), :],
                             xbuf.at[1 - p], sem)

        pv = jnp.full((L,), 1, jnp.int32) * p

        sm1 = [negv] * N_SUPER
        sm2 = [negv] * N_SUPER
        for k in range(N_EXPERTS):
            si = k % N_SUPER
            e = si * L + k // N_SUPER
            ev = jnp.full((L,), e, jnp.int32)
            s = plsc.load_gather(xbuf, [pv, lane, ev])
            gt = s > sm1[si]
            sm2[si] = jnp.where(gt, sm1[si], jnp.maximum(sm2[si], s))
            sm1[si] = jnp.maximum(sm1[si], s)

        g_sc = []
        for g in range(N_GROUP):
            a1, a2 = sm1[2 * g], sm2[2 * g]
            b1, b2 = sm1[2 * g + 1], sm2[2 * g + 1]
            hi = jnp.maximum(a1, b1)
            lo = jnp.minimum(a1, b1)
            sec = jnp.maximum(lo, jnp.where(a1 > b1, a2, b2))
            g_sc.append(hi + sec)

        chosen = [None] * N_GROUP
        for _ in range(TOPK_GROUP):
            best = g_sc[0]
            for g in range(1, N_GROUP):
                best = jnp.maximum(best, g_sc[g])
            found = lane < 0
            for g in range(N_GROUP):
                hit = (g_sc[g] == best) & (~found)
                chosen[g] = hit if chosen[g] is None else (chosen[g] | hit)
                found = found | hit
                g_sc[g] = jnp.where(hit, negv, g_sc[g])

        for s_i in range(N_SUPER):
            m = jnp.where(chosen[s_i // SUP_PER_GROUP], sm1[s_i], negv)
            sup[pl.ds(s_i * L, L)] = m

        def round_body(r, wsum):
            m = negv
            mi = zero_i
            for s_i in range(N_SUPER):
                v = sup[pl.ds(s_i * L, L)]
                gt = v > m
                m = jnp.where(gt, v, m)
                mi = jnp.where(gt, jnp.full((L,), s_i, jnp.int32), mi)
            miL = mi * L
            lm = negv
            lj = zero_i
            for j in range(L):
                v = plsc.load_gather(xbuf, [pv, lane, miL + j])
                gt = v > lm
                lm = jnp.where(gt, v, lm)
                lj = jnp.where(gt, jnp.full((L,), j, jnp.int32), lj)
            eidx = miL + lj
            plsc.store_scatter(xbuf, [pv, lane, eidx], negv)
            nm = negv
            for j in range(L):
                nm = jnp.maximum(nm,
                                 plsc.load_gather(xbuf, [pv, lane, miL + j]))
            plsc.store_scatter(sup, [miL + lane], nm)
            b = plsc.load_gather(bias_v, [eidx])
            w = lm - b
            tok = t * L + lane
            rv = jnp.full((L,), 1, jnp.int32) * r
            plsc.store_scatter(oi, [tok, rv], eidx)
            plsc.store_scatter(ow, [tok, rv], w)
            return wsum + w

        wsum = lax.fori_loop(0, TOP_K, round_body,
                             jnp.zeros((L,), jnp.float32))
        scale = SCALING / (wsum + 1e-20)
        tok = t * L + lane
        for r in range(TOP_K):
            rv = jnp.full((L,), r, jnp.int32)
            w = plsc.load_gather(ow, [tok, rv])
            plsc.store_scatter(ow, [tok, rv], w * scale)
        return 0

    lax.fori_loop(0, n_tiles, tile_body, 0)
    pltpu.sync_copy(oi, idx_hbm.at[pl.ds(wid * tok_per_w, tok_per_w), :])
    pltpu.sync_copy(ow, w_hbm.at[pl.ds(wid * tok_per_w, tok_per_w), :])


def kernel(router_logits, correction_bias):
    scores = jax.nn.sigmoid(router_logits).astype(jnp.float32)
    adj = scores + correction_bias[None, :]
    n_tokens = router_logits.shape[0]
    tok_per_w = n_tokens // NW
    mesh = plsc.VectorSubcoreMesh(core_axis_name="c", subcore_axis_name="s",
                                  num_cores=NC, num_subcores=NS)
    run = pl.kernel(
        _router_body,
        out_type=(
            jax.ShapeDtypeStruct((n_tokens, TOP_K), jnp.int32),
            jax.ShapeDtypeStruct((n_tokens, TOP_K), jnp.float32),
        ),
        mesh=mesh,
        scratch_types=[
            pltpu.VMEM((2, L, N_EXPERTS), jnp.float32),
            pltpu.VMEM((N_SUPER * L,), jnp.float32),
            pltpu.VMEM((N_EXPERTS,), jnp.float32),
            pltpu.VMEM((tok_per_w, TOP_K), jnp.int32),
            pltpu.VMEM((tok_per_w, TOP_K), jnp.float32),
            pltpu.SemaphoreType.DMA,
        ],
        compiler_params=pltpu.CompilerParams(use_tc_tiling_on_sc=False,
                                             needs_layout_passes=False),
    )
    return run(adj, correction_bias)


# speedup vs baseline: 65.7471x; 1.0806x over previous
import jax
import jax.numpy as jnp
from jax import lax
from jax.experimental import pallas as pl
from jax.experimental.pallas import tpu as pltpu
from jax.experimental.pallas import tpu_sc as plsc

TOP_K = 8
N_EXPERTS = 256
SCALING = 2.5
N_GROUP = 8
TOPK_GROUP = 4
GROUP_SIZE = N_EXPERTS // N_GROUP

NC, NS, L = 2, 16, 16
NW = NC * NS
N_SUPER = N_EXPERTS // L
SUP_PER_GROUP = GROUP_SIZE // L
NEG = -1e9
TILE_ELEMS = L * N_EXPERTS


def _router_body(adj_hbm, bias_hbm, idx_hbm, w_hbm, xbuf, sup, bias_v, oi, ow,
                 sem):
    n_tokens = adj_hbm.shape[0] // N_EXPERTS
    tok_per_w = n_tokens // NW
    n_tiles = tok_per_w // L

    wid = lax.axis_index("s") * NC + lax.axis_index("c")
    lane = lax.iota(jnp.int32, L)
    lane_row = lane * N_EXPERTS
    lane8 = lane * TOP_K
    negv = jnp.full((L,), NEG, jnp.float32)
    zero_i = jnp.zeros((L,), jnp.int32)

    pltpu.sync_copy(bias_hbm, bias_v)
    in_base = wid * tok_per_w * N_EXPERTS
    pltpu.async_copy(adj_hbm.at[pl.ds(in_base, TILE_ELEMS)],
                     xbuf.at[pl.ds(0, TILE_ELEMS)], sem)

    def tile_body(t, _):
        p = lax.rem(t, 2)
        pbase = p * TILE_ELEMS
        pltpu.make_async_copy(
            adj_hbm.at[pl.ds(in_base + t * TILE_ELEMS, TILE_ELEMS)],
            xbuf.at[pl.ds(pbase, TILE_ELEMS)], sem).wait()

        @pl.when(t < n_tiles - 1)
        def _prefetch():
            pltpu.async_copy(
                adj_hbm.at[pl.ds(in_base + (t + 1) * TILE_ELEMS, TILE_ELEMS)],
                xbuf.at[pl.ds(TILE_ELEMS - pbase, TILE_ELEMS)], sem)

        vb = pbase + lane_row

        sm1 = [negv] * N_SUPER
        sm2 = [negv] * N_SUPER
        for k in range(N_EXPERTS):
            si = k % N_SUPER
            e = si * L + k // N_SUPER
            s = plsc.load_gather(xbuf, [vb + e])
            gt = s > sm1[si]
            sm2[si] = jnp.where(gt, sm1[si], jnp.maximum(sm2[si], s))
            sm1[si] = jnp.maximum(sm1[si], s)

        g_sc = []
        for g in range(N_GROUP):
            a1, a2 = sm1[2 * g], sm2[2 * g]
            b1, b2 = sm1[2 * g + 1], sm2[2 * g + 1]
            hi = jnp.maximum(a1, b1)
            lo = jnp.minimum(a1, b1)
            sec = jnp.maximum(lo, jnp.where(a1 > b1, a2, b2))
            g_sc.append(hi + sec)

        chosen = [None] * N_GROUP
        for _ in range(TOPK_GROUP):
            best = g_sc[0]
            for g in range(1, N_GROUP):
                best = jnp.maximum(best, g_sc[g])
            found = lane < 0
            for g in range(N_GROUP):
                hit = (g_sc[g] == best) & (~found)
                chosen[g] = hit if chosen[g] is None else (chosen[g] | hit)
                found = found | hit
                g_sc[g] = jnp.where(hit, negv, g_sc[g])

        for s_i in range(N_SUPER):
            sup[pl.ds(s_i * L, L)] = jnp.where(chosen[s_i // SUP_PER_GROUP],
                                               sm1[s_i], negv)

        wsum = jnp.zeros((L,), jnp.float32)
        obase = t * L * TOP_K
        for r in range(TOP_K):
            m = negv
            mi = zero_i
            for s_i in range(N_SUPER):
                v = sup[pl.ds(s_i * L, L)]
                gt = v > m
                m = jnp.where(gt, v, m)
                mi = jnp.where(gt, jnp.full((L,), s_i, jnp.int32), mi)
            lbase = vb + mi * L
            lm = negv
            lm2 = negv
            lj = zero_i
            for j in range(L):
                v = plsc.load_gather(xbuf, [lbase + j])
                gt = v > lm
                lm2 = jnp.where(gt, lm, jnp.maximum(lm2, v))
                lm = jnp.maximum(lm, v)
                lj = jnp.where(gt, jnp.full((L,), j, jnp.int32), lj)
            eidx = mi * L + lj
            plsc.store_scatter(xbuf, [lbase + lj], negv)
            plsc.store_scatter(sup, [mi * L + lane], lm2)
            b = plsc.load_gather(bias_v, [eidx])
            w = lm - b
            plsc.store_scatter(oi, [lane8 + (obase + r)], eidx)
            plsc.store_scatter(ow, [lane8 + (obase + r)], w)
            wsum = wsum + w

        scale = SCALING / (wsum + 1e-20)
        for r in range(TOP_K):
            oaddr = lane8 + (obase + r)
            w = plsc.load_gather(ow, [oaddr])
            plsc.store_scatter(ow, [oaddr], w * scale)
        return 0

    lax.fori_loop(0, n_tiles, tile_body, 0)
    out_base = wid * tok_per_w * TOP_K
    out_len = tok_per_w * TOP_K
    pltpu.sync_copy(oi, idx_hbm.at[pl.ds(out_base, out_len)])
    pltpu.sync_copy(ow, w_hbm.at[pl.ds(out_base, out_len)])


def kernel(router_logits, correction_bias):
    scores = jax.nn.sigmoid(router_logits).astype(jnp.float32)
    adj = scores + correction_bias[None, :]
    n_tokens = router_logits.shape[0]
    tok_per_w = n_tokens // NW
    mesh = plsc.VectorSubcoreMesh(core_axis_name="c", subcore_axis_name="s",
                                  num_cores=NC, num_subcores=NS)
    run = pl.kernel(
        _router_body,
        out_type=(
            jax.ShapeDtypeStruct((n_tokens * TOP_K,), jnp.int32),
            jax.ShapeDtypeStruct((n_tokens * TOP_K,), jnp.float32),
        ),
        mesh=mesh,
        scratch_types=[
            pltpu.VMEM((2 * TILE_ELEMS,), jnp.float32),
            pltpu.VMEM((N_SUPER * L,), jnp.float32),
            pltpu.VMEM((N_EXPERTS,), jnp.float32),
            pltpu.VMEM((tok_per_w * TOP_K,), jnp.int32),
            pltpu.VMEM((tok_per_w * TOP_K,), jnp.float32),
            pltpu.SemaphoreType.DMA,
        ],
        compiler_params=pltpu.CompilerParams(use_tc_tiling_on_sc=False,
                                             needs_layout_passes=False),
    )
    idx_flat, w_flat = run(adj.reshape(n_tokens * N_EXPERTS), correction_bias)
    return (idx_flat.reshape(n_tokens, TOP_K),
            w_flat.reshape(n_tokens, TOP_K))
